# Initial kernel scaffold; baseline (speedup 1.0000x reference)
import jax
import jax.numpy as jnp
from jax.experimental import pallas as pl


def kernel(x, bn0_gamma, bn0_beta, W0, b0, g0, be0, W1, b1, g1, be1, W2, b2, g2, be2):
    raise NotImplementedError("write your pallas kernel here")


# speedup vs baseline: 5.2352x; 5.2352x over previous
import functools

import jax
import jax.numpy as jnp
import numpy as np
from jax.experimental import pallas as pl

_S = 128
_NPAIR = _S * (_S + 1) // 2
_TWO_PI = 2.0 * np.pi


def _tril_mask():
    r = jax.lax.broadcasted_iota(jnp.int32, (_S, _S), 0)
    c = jax.lax.broadcasted_iota(jnp.int32, (_S, _S), 1)
    return (c <= r).astype(jnp.float32)


def _ptrapphi(px, py, pz, e):
    rap = 0.5 * jnp.log(1.0 + 2.0 * pz / jnp.maximum(e - pz, 1e-20))
    phi = jnp.arctan2(py, px)
    return rap, phi


def _pair_feats(x_row, x_col):
    px_r, py_r, pz_r, e_r = (x_row[k:k + 1, :] for k in range(4))
    px_c, py_c, pz_c, e_c = (x_col[:, k:k + 1] for k in range(4))
    rap_r, phi_r = _ptrapphi(px_r, py_r, pz_r, e_r)
    rap_c, phi_c = _ptrapphi(px_c, py_c, pz_c, e_c)
    d = phi_c - phi_r + np.pi
    dphi = d - _TWO_PI * jnp.floor(d / _TWO_PI) - np.pi
    drap = rap_c - rap_r
    delta = jnp.sqrt(drap * drap + dphi * dphi)
    lndelta = jnp.log(jnp.maximum(delta, 1e-8))
    pxs = px_c + px_r
    pys = py_c + py_r
    pzs = pz_c + pz_r
    es = e_c + e_r
    m2 = jnp.maximum(es * es - (pxs * pxs + pys * pys + pzs * pzs), 1e-8)
    lnm2 = jnp.log(m2)
    return lndelta, lnm2


def _k0(x_ref, xt_ref, st_ref):
    f0, f1 = _pair_feats(x_ref[0], xt_ref[0])
    m = _tril_mask()
    st_ref[0, 0:1, :] = jnp.sum(f0 * m, axis=0, keepdims=True)
    st_ref[0, 1:2, :] = jnp.sum(f1 * m, axis=0, keepdims=True)
    st_ref[0, 2:3, :] = jnp.sum(f0 * f0 * m, axis=0, keepdims=True)
    st_ref[0, 3:4, :] = jnp.sum(f1 * f1 * m, axis=0, keepdims=True)


def _k1(x_ref, xt_ref, A_ref, c_ref, zo_ref, ss_ref, sq_ref):
    f0, f1 = _pair_feats(x_ref[0], xt_ref[0])
    A = A_ref[...]
    a0 = jnp.reshape(A[:, 0:1], (64, 1, 1))
    a1 = jnp.reshape(A[:, 1:2], (64, 1, 1))
    cc = jnp.reshape(c_ref[...], (64, 1, 1))
    z = a0 * f0[None, :, :] + a1 * f1[None, :, :] + cc
    zo_ref[0] = z
    m = _tril_mask()[None, :, :]
    ss_ref[0] = jnp.sum(z * m, axis=2)
    sq_ref[0] = jnp.sum(z * z * m, axis=2)


def _klayer(z_ref, a_ref, c_ref, w_ref, b_ref, zo_ref, ss_ref, sq_ref):
    a = jnp.reshape(a_ref[...], (64, 1, 1))
    c = jnp.reshape(c_ref[...], (64, 1, 1))
    h = jnp.maximum(a * z_ref[0] + c, 0.0)
    hf = jnp.reshape(h, (64, _S * _S))
    z = jax.lax.dot_general(w_ref[...], hf, (((1,), (0,)), ((), ())),
                            preferred_element_type=jnp.float32)
    z = z + b_ref[...]
    z3 = jnp.reshape(z, (64, _S, _S))
    zo_ref[0] = z3
    m = _tril_mask()[None, :, :]
    ss_ref[0] = jnp.sum(z3 * m, axis=2)
    sq_ref[0] = jnp.sum(z3 * z3 * m, axis=2)


def _k4(z_ref, a_ref, c_ref, y_ref):
    a = jnp.reshape(a_ref[...], (64, 1, 1))
    c = jnp.reshape(c_ref[...], (64, 1, 1))
    y_ref[0] = jnp.maximum(a * z_ref[0] + c, 0.0)


def _const_spec(shape):
    return pl.BlockSpec(shape, lambda b: tuple(0 for _ in shape))


def _stats_to_affine(s, q, g, be, n):
    mean = s / n
    var = q / n - mean * mean
    inv = jax.lax.rsqrt(var + 1e-5)
    a = g * inv
    c = be - mean * a
    return a, c


@jax.jit
def kernel(x, bn0_gamma, bn0_beta, W0, b0, g0, be0, W1, b1, g1, be1,
           W2, b2, g2, be2):
    B = x.shape[0]
    xt = jnp.transpose(x, (0, 2, 1))
    n = float(_NPAIR * B)
    f32 = jnp.float32

    x_spec = pl.BlockSpec((1, 4, _S), lambda b: (b, 0, 0))
    xt_spec = pl.BlockSpec((1, _S, 4), lambda b: (b, 0, 0))
    z_spec = pl.BlockSpec((1, 64, _S, _S), lambda b: (b, 0, 0, 0))
    st_spec = pl.BlockSpec((1, 64, _S), lambda b: (b, 0, 0))
    z_shape = jax.ShapeDtypeStruct((B, 64, _S, _S), f32)
    st_shape = jax.ShapeDtypeStruct((B, 64, _S), f32)

    st0 = pl.pallas_call(
        _k0,
        grid=(B,),
        in_specs=[x_spec, xt_spec],
        out_specs=pl.BlockSpec((1, 4, _S), lambda b: (b, 0, 0)),
        out_shape=jax.ShapeDtypeStruct((B, 4, _S), f32),
    )(x, xt)
    s = jnp.sum(st0[:, 0:2, :], axis=(0, 2))
    q = jnp.sum(st0[:, 2:4, :], axis=(0, 2))
    a0, c0 = _stats_to_affine(s, q, bn0_gamma, bn0_beta, n)
    A = W0 * a0[None, :]
    cv = (W0 @ c0 + b0).reshape(64, 1)

    z1, ss, sq = pl.pallas_call(
        _k1,
        grid=(B,),
        in_specs=[x_spec, xt_spec, _const_spec((64, 2)), _const_spec((64, 1))],
        out_specs=[z_spec, st_spec, st_spec],
        out_shape=[z_shape, st_shape, st_shape],
    )(x, xt, A, cv)

    zs = z1
    params = ((g0, be0, W1, b1), (g1, be1, W2, b2))
    layer_call = pl.pallas_call(
        _klayer,
        grid=(B,),
        in_specs=[z_spec, _const_spec((64, 1)), _const_spec((64, 1)),
                  _const_spec((64, 64)), _const_spec((64, 1))],
        out_specs=[z_spec, st_spec, st_spec],
        out_shape=[z_shape, st_shape, st_shape],
    )
    for g, be, W, b in params:
        a, c = _stats_to_affine(jnp.sum(ss, axis=(0, 2)),
                                jnp.sum(sq, axis=(0, 2)), g, be, n)
        zs, ss, sq = layer_call(zs, a.reshape(64, 1), c.reshape(64, 1),
                                W, b.reshape(64, 1))

    a, c = _stats_to_affine(jnp.sum(ss, axis=(0, 2)),
                            jnp.sum(sq, axis=(0, 2)), g2, be2, n)
    y = pl.pallas_call(
        _k4,
        grid=(B,),
        in_specs=[z_spec, _const_spec((64, 1)), _const_spec((64, 1))],
        out_specs=z_spec,
        out_shape=z_shape,
    )(zs, a.reshape(64, 1), c.reshape(64, 1))
    return y
